# baseline (speedup 1.0000x reference)
import functools

import jax
import jax.numpy as jnp
from jax import lax
from jax.experimental import pallas as pl
from jax.experimental.pallas import tpu as pltpu
from jax.experimental.pallas import tpu_sc as plsc

_NUM_CORES = 2
_NUM_SUBCORES = 16
_NUM_WORKERS = _NUM_CORES * _NUM_SUBCORES
_LANES = 16


def _build_sc_kernel(num_loc, K, D):
    per_w = num_loc // _NUM_WORKERS
    n_groups = per_w // _LANES
    n_chunks = per_w // 128
    sims_words = per_w * K

    mesh = plsc.VectorSubcoreMesh(core_axis_name="c", subcore_axis_name="s")

    @functools.partial(
        pl.kernel,
        mesh=mesh,
        compiler_params=pltpu.CompilerParams(
            needs_layout_passes=False, use_tc_tiling_on_sc=False
        ),
        out_type=jax.ShapeDtypeStruct((num_loc, D + 1), jnp.float32),
        scratch_types=[
            pltpu.VMEM((sims_words,), jnp.float32),
            pltpu.VMEM((n_chunks, 128), jnp.int32),
            pltpu.VMEM((per_w, D), jnp.float32),
            pltpu.VMEM((per_w, D + 1), jnp.float32),
            pltpu.SemaphoreType.DMA,
        ],
    )
    def body(dist_hbm, sims_hbm, out_hbm, sims_v, idx_v, rows_v, outbuf_v, sem):
        wid = lax.axis_index("s") * _NUM_CORES + lax.axis_index("c")
        base_loc = wid * per_w
        iota = lax.iota(jnp.int32, _LANES)

        pltpu.sync_copy(sims_hbm.at[pl.ds(base_loc * K, sims_words)], sims_v)

        def group_body(g, carry):
            gidx = g * (_LANES * K) + iota * K
            best_val = plsc.load_gather(sims_v, [gidx])
            best_k = jnp.zeros((_LANES,), jnp.int32)
            for k in range(1, K):
                v = plsc.load_gather(sims_v, [gidx + k])
                take = v > best_val
                best_val = jnp.where(take, v, best_val)
                best_k = jnp.where(take, k, best_k)
            l0 = g * _LANES + iota
            rowids = (base_loc + l0) * K + best_k
            plsc.store_scatter(idx_v, [l0 // 128, l0 % 128], rowids)
            plsc.store_scatter(
                outbuf_v,
                [l0, jnp.full((_LANES,), D, jnp.int32)],
                best_k.astype(jnp.float32),
            )
            return carry

        lax.fori_loop(0, n_groups, group_body, 0)

        copies = [
            pltpu.async_copy(
                dist_hbm.at[idx_v.at[c]],
                rows_v.at[pl.ds(c * 128, 128)],
                sem,
            )
            for c in range(n_chunks)
        ]
        for cp in copies:
            cp.wait()

        def copy_body(l, carry):
            for u in range(4):
                ll = l * 4 + u
                outbuf_v[ll, pl.ds(0, D)] = rows_v[ll]
            return carry

        lax.fori_loop(0, per_w // 4, copy_body, 0)

        pltpu.sync_copy(outbuf_v, out_hbm.at[pl.ds(base_loc, per_w)])

    return body


def kernel(distance, similarities):
    B, i, j, K, D = distance.shape
    num_loc = B * i * j
    dist2d = distance.reshape(num_loc * K, D)
    sims1d = similarities.reshape(num_loc * K)
    out = _build_sc_kernel(num_loc, K, D)(dist2d, sims1d)
    return out.reshape(B, i * j, D + 1)


# speedup vs baseline: 3.6876x; 3.6876x over previous
import functools

import jax
import jax.numpy as jnp
from jax import lax
from jax.experimental import pallas as pl
from jax.experimental.pallas import tpu as pltpu
from jax.experimental.pallas import tpu_sc as plsc

_NUM_CORES = 2
_NUM_SUBCORES = 16
_NUM_WORKERS = _NUM_CORES * _NUM_SUBCORES
_LANES = 16
_CHUNK = 16
_SSTAGE = 128


def _build_sc_kernel(num_loc, K, D):
    per_w = num_loc // _NUM_WORKERS
    n_sstages = per_w // _SSTAGE
    n_chunks = per_w // _CHUNK
    out_row = D + 1

    mesh = plsc.VectorSubcoreMesh(core_axis_name="c", subcore_axis_name="s")

    @functools.partial(
        pl.kernel,
        mesh=mesh,
        compiler_params=pltpu.CompilerParams(needs_layout_passes=False),
        out_type=jax.ShapeDtypeStruct((num_loc * out_row,), jnp.float32),
        scratch_types=[
            pltpu.VMEM((_SSTAGE, K), jnp.float32),
            pltpu.VMEM((per_w,), jnp.int32),
            pltpu.VMEM((_CHUNK, D, K), jnp.float32),
            pltpu.VMEM((_CHUNK, D, K), jnp.float32),
            pltpu.VMEM((per_w * out_row,), jnp.float32),
            pltpu.SemaphoreType.DMA,
            pltpu.SemaphoreType.DMA,
        ],
    )
    def body(dist_hbm, sims_hbm, out_hbm, sims_v, kbuf_v, db0, db1, outbuf_v,
             sem0, sem1):
        wid = lax.axis_index("s") * _NUM_CORES + lax.axis_index("c")
        base_loc = wid * per_w
        iota = lax.iota(jnp.int32, _LANES)
        dbufs = (db0, db1)
        sems = (sem0, sem1)

        for s in range(n_sstages):
            pltpu.sync_copy(
                sims_hbm.at[pl.ds(base_loc + s * _SSTAGE, _SSTAGE)], sims_v
            )

            def group_body(g, carry, s=s):
                l0 = g * _LANES + iota
                best_val = plsc.load_gather(
                    sims_v, [l0, jnp.zeros((_LANES,), jnp.int32)]
                )
                best_k = jnp.zeros((_LANES,), jnp.int32)
                for k in range(1, K):
                    v = plsc.load_gather(
                        sims_v, [l0, jnp.full((_LANES,), k, jnp.int32)]
                    )
                    take = v > best_val
                    best_val = jnp.where(take, v, best_val)
                    best_k = jnp.where(take, k, best_k)
                gl = s * _SSTAGE + l0
                plsc.store_scatter(kbuf_v, [gl], best_k)
                plsc.store_scatter(
                    outbuf_v, [gl * out_row + D], best_k.astype(jnp.float32)
                )
                return carry

            lax.fori_loop(0, _SSTAGE // _LANES, group_body, 0)

        def start(c):
            return pltpu.async_copy(
                dist_hbm.at[pl.ds(base_loc + c * _CHUNK, _CHUNK)],
                dbufs[c % 2],
                sems[c % 2],
            )

        pending = start(0)
        for c in range(n_chunks):
            nxt = start(c + 1) if c + 1 < n_chunks else None
            pending.wait()
            _extract(dbufs[c % 2], kbuf_v, outbuf_v, c * _CHUNK, D, out_row,
                     iota)
            pending = nxt

        pltpu.sync_copy(
            outbuf_v, out_hbm.at[pl.ds(base_loc * out_row, per_w * out_row)]
        )

    return body


def _extract(dbuf, kbuf_v, outbuf_v, loc_base, D, out_row, iota):
    for g in range(_CHUNK // _LANES):
        lb = loc_base + g * _LANES
        ks = kbuf_v[pl.ds(lb, _LANES)]
        jvec = g * _LANES + iota
        opos = (lb + iota) * out_row
        for dd in range(D):
            val = plsc.load_gather(
                dbuf, [jvec, jnp.full((_LANES,), dd, jnp.int32), ks]
            )
            plsc.store_scatter(outbuf_v, [opos + dd], val)


def kernel(distance, similarities):
    B, i, j, K, D = distance.shape
    num_loc = B * i * j
    dist_t = jnp.transpose(distance, (0, 1, 2, 4, 3)).reshape(num_loc, D, K)
    sims2d = similarities.reshape(num_loc, K)
    out = _build_sc_kernel(num_loc, K, D)(dist_t, sims2d)
    return out.reshape(B, i * j, D + 1)


# speedup vs baseline: 5.6117x; 1.5218x over previous
import functools

import jax
import jax.numpy as jnp
from jax import lax
from jax.experimental import pallas as pl
from jax.experimental.pallas import tpu as pltpu
from jax.experimental.pallas import tpu_sc as plsc

_NUM_CORES = 2
_NUM_SUBCORES = 16
_NUM_WORKERS = _NUM_CORES * _NUM_SUBCORES
_LANES = 16
_CHUNK = 16
_SC_SHARE = 6144
_TC_BLK = 512


def _build_sc_kernel(num_loc, K, D, sc_loc):
    per_w = sc_loc // _NUM_WORKERS
    n_chunks = per_w // _CHUNK
    out_row = D + 1

    mesh = plsc.VectorSubcoreMesh(core_axis_name="c", subcore_axis_name="s")

    @functools.partial(
        pl.kernel,
        mesh=mesh,
        compiler_params=pltpu.CompilerParams(needs_layout_passes=False),
        out_type=jax.ShapeDtypeStruct((sc_loc * out_row,), jnp.float32),
        scratch_types=[
            pltpu.VMEM((per_w, K), jnp.float32),
            pltpu.VMEM((per_w,), jnp.int32),
            pltpu.VMEM((_CHUNK, D, K), jnp.float32),
            pltpu.VMEM((_CHUNK, D, K), jnp.float32),
            pltpu.VMEM((per_w * out_row,), jnp.float32),
            pltpu.SemaphoreType.DMA,
            pltpu.SemaphoreType.DMA,
        ],
    )
    def body(dist_hbm, sims_hbm, out_hbm, sims_v, kbuf_v, db0, db1, outbuf_v,
             sem0, sem1):
        wid = lax.axis_index("s") * _NUM_CORES + lax.axis_index("c")
        base_loc = wid * per_w
        iota = lax.iota(jnp.int32, _LANES)
        dbufs = (db0, db1)
        sems = (sem0, sem1)

        def start(c):
            return pltpu.async_copy(
                dist_hbm.at[pl.ds(base_loc + c * _CHUNK, _CHUNK)],
                dbufs[c % 2],
                sems[c % 2],
            )

        pending = start(0)

        pltpu.sync_copy(sims_hbm.at[pl.ds(base_loc, per_w)], sims_v)

        def group_body(g, carry):
            l0 = g * _LANES + iota
            best_val = plsc.load_gather(
                sims_v, [l0, jnp.zeros((_LANES,), jnp.int32)]
            )
            best_k = jnp.zeros((_LANES,), jnp.int32)
            for k in range(1, K):
                v = plsc.load_gather(
                    sims_v, [l0, jnp.full((_LANES,), k, jnp.int32)]
                )
                take = v > best_val
                best_val = jnp.where(take, v, best_val)
                best_k = jnp.where(take, k, best_k)
            plsc.store_scatter(kbuf_v, [l0], best_k)
            plsc.store_scatter(
                outbuf_v, [l0 * out_row + D], best_k.astype(jnp.float32)
            )
            return carry

        lax.fori_loop(0, per_w // _LANES, group_body, 0)

        for c in range(n_chunks):
            nxt = start(c + 1) if c + 1 < n_chunks else None
            pending.wait()
            _extract(dbufs[c % 2], kbuf_v, outbuf_v, c * _CHUNK, D, out_row,
                     iota)
            pending = nxt

        pltpu.sync_copy(
            outbuf_v, out_hbm.at[pl.ds(base_loc * out_row, per_w * out_row)]
        )

    return body


def _extract(dbuf, kbuf_v, outbuf_v, loc_base, D, out_row, iota):
    for g in range(_CHUNK // _LANES):
        lb = loc_base + g * _LANES
        ks = kbuf_v[pl.ds(lb, _LANES)]
        jvec = g * _LANES + iota
        opos = (lb + iota) * out_row
        for dd in range(D):
            val = plsc.load_gather(
                dbuf, [jvec, jnp.full((_LANES,), dd, jnp.int32), ks]
            )
            plsc.store_scatter(outbuf_v, [opos + dd], val)


def _tc_body(K, D, d_ref, s_ref, o_ref):
    s = s_ref[...]
    ik = lax.broadcasted_iota(jnp.int32, s.shape, 1)
    m = jnp.max(s, axis=-1, keepdims=True)
    am = jnp.min(jnp.where(s == m, ik, K), axis=-1)
    onehot = (ik == am[:, None]).astype(jnp.float32)
    d = d_ref[...]
    resid = jnp.sum(d * onehot[:, None, :], axis=-1)
    o_ref[...] = jnp.concatenate(
        [resid, am[:, None].astype(jnp.float32)], axis=-1
    )


def _tc_kernel(dist_t, sims2d, start_loc):
    num_loc, D, K = dist_t.shape
    n = num_loc - start_loc
    off = start_loc // _TC_BLK
    return pl.pallas_call(
        functools.partial(_tc_body, K, D),
        grid=(n // _TC_BLK,),
        in_specs=[
            pl.BlockSpec((_TC_BLK, D, K), lambda g: (g + off, 0, 0)),
            pl.BlockSpec((_TC_BLK, K), lambda g: (g + off, 0)),
        ],
        out_specs=pl.BlockSpec((_TC_BLK, D + 1), lambda g: (g, 0)),
        out_shape=jax.ShapeDtypeStruct((n, D + 1), jnp.float32),
    )(dist_t, sims2d)


def kernel(distance, similarities):
    B, i, j, K, D = distance.shape
    num_loc = B * i * j
    dist_t = jnp.transpose(distance, (0, 1, 2, 4, 3)).reshape(num_loc, D, K)
    sims2d = similarities.reshape(num_loc, K)
    sc_out = _build_sc_kernel(num_loc, K, D, _SC_SHARE)(dist_t, sims2d)
    tc_out = _tc_kernel(dist_t, sims2d, _SC_SHARE)
    out = jnp.concatenate([sc_out.reshape(_SC_SHARE, D + 1), tc_out], axis=0)
    return out.reshape(B, i * j, D + 1)


# speedup vs baseline: 5.7315x; 1.0213x over previous
import functools

import jax
import jax.numpy as jnp
from jax import lax
from jax.experimental import pallas as pl
from jax.experimental.pallas import tpu as pltpu
from jax.experimental.pallas import tpu_sc as plsc

_NUM_CORES = 2
_NUM_SUBCORES = 16
_NUM_WORKERS = _NUM_CORES * _NUM_SUBCORES
_LANES = 16
_CHUNK = 16
_SC_SHARE = 7168
_TC_BLK = 512


def _build_sc_kernel(num_loc, K, D, sc_loc):
    per_w = sc_loc // _NUM_WORKERS
    n_chunks = per_w // _CHUNK
    out_row = D + 1

    mesh = plsc.VectorSubcoreMesh(core_axis_name="c", subcore_axis_name="s")

    @functools.partial(
        pl.kernel,
        mesh=mesh,
        compiler_params=pltpu.CompilerParams(needs_layout_passes=False),
        out_type=jax.ShapeDtypeStruct((sc_loc * out_row,), jnp.float32),
        scratch_types=[
            pltpu.VMEM((per_w, K), jnp.float32),
            pltpu.VMEM((per_w,), jnp.int32),
            pltpu.VMEM((_CHUNK, D, K), jnp.float32),
            pltpu.VMEM((_CHUNK, D, K), jnp.float32),
            pltpu.VMEM((per_w * out_row,), jnp.float32),
            pltpu.SemaphoreType.DMA,
            pltpu.SemaphoreType.DMA,
        ],
    )
    def body(dist_hbm, sims_hbm, out_hbm, sims_v, kbuf_v, db0, db1, outbuf_v,
             sem0, sem1):
        wid = lax.axis_index("s") * _NUM_CORES + lax.axis_index("c")
        base_loc = wid * per_w
        iota = lax.iota(jnp.int32, _LANES)
        dbufs = (db0, db1)
        sems = (sem0, sem1)

        def start(c):
            return pltpu.async_copy(
                dist_hbm.at[pl.ds(base_loc + c * _CHUNK, _CHUNK)],
                dbufs[c % 2],
                sems[c % 2],
            )

        pending = start(0)

        pltpu.sync_copy(sims_hbm.at[pl.ds(base_loc, per_w)], sims_v)

        def group_body(g, carry):
            l0 = g * _LANES + iota
            best_val = plsc.load_gather(
                sims_v, [l0, jnp.zeros((_LANES,), jnp.int32)]
            )
            best_k = jnp.zeros((_LANES,), jnp.int32)
            for k in range(1, K):
                v = plsc.load_gather(
                    sims_v, [l0, jnp.full((_LANES,), k, jnp.int32)]
                )
                take = v > best_val
                best_val = jnp.where(take, v, best_val)
                best_k = jnp.where(take, k, best_k)
            plsc.store_scatter(kbuf_v, [l0], best_k)
            plsc.store_scatter(
                outbuf_v, [l0 * out_row + D], best_k.astype(jnp.float32)
            )
            return carry

        lax.fori_loop(0, per_w // _LANES, group_body, 0)

        for c in range(n_chunks):
            nxt = start(c + 1) if c + 1 < n_chunks else None
            pending.wait()
            _extract(dbufs[c % 2], kbuf_v, outbuf_v, c * _CHUNK, D, out_row,
                     iota)
            pending = nxt

        pltpu.sync_copy(
            outbuf_v, out_hbm.at[pl.ds(base_loc * out_row, per_w * out_row)]
        )

    return body


def _extract(dbuf, kbuf_v, outbuf_v, loc_base, D, out_row, iota):
    for g in range(_CHUNK // _LANES):
        lb = loc_base + g * _LANES
        ks = kbuf_v[pl.ds(lb, _LANES)]
        jvec = g * _LANES + iota
        opos = (lb + iota) * out_row
        for dd in range(D):
            val = plsc.load_gather(
                dbuf, [jvec, jnp.full((_LANES,), dd, jnp.int32), ks]
            )
            plsc.store_scatter(outbuf_v, [opos + dd], val)


def _tc_body(K, D, d_ref, s_ref, o_ref):
    s = s_ref[...]
    ik = lax.broadcasted_iota(jnp.int32, s.shape, 1)
    m = jnp.max(s, axis=-1, keepdims=True)
    am = jnp.min(jnp.where(s == m, ik, K), axis=-1)
    onehot = (ik == am[:, None]).astype(jnp.float32)
    d = d_ref[...]
    resid = jnp.sum(d * onehot[:, None, :], axis=-1)
    o_ref[...] = jnp.concatenate(
        [resid, am[:, None].astype(jnp.float32)], axis=-1
    )


def _tc_kernel(dist_t, sims2d, start_loc):
    num_loc, D, K = dist_t.shape
    n = num_loc - start_loc
    off = start_loc // _TC_BLK
    return pl.pallas_call(
        functools.partial(_tc_body, K, D),
        grid=(n // _TC_BLK,),
        in_specs=[
            pl.BlockSpec((_TC_BLK, D, K), lambda g: (g + off, 0, 0)),
            pl.BlockSpec((_TC_BLK, K), lambda g: (g + off, 0)),
        ],
        out_specs=pl.BlockSpec((_TC_BLK, D + 1), lambda g: (g, 0)),
        out_shape=jax.ShapeDtypeStruct((n, D + 1), jnp.float32),
    )(dist_t, sims2d)


def kernel(distance, similarities):
    B, i, j, K, D = distance.shape
    num_loc = B * i * j
    dist_t = jnp.transpose(distance, (0, 1, 2, 4, 3)).reshape(num_loc, D, K)
    sims2d = similarities.reshape(num_loc, K)
    sc_out = _build_sc_kernel(num_loc, K, D, _SC_SHARE)(dist_t, sims2d)
    tc_out = _tc_kernel(dist_t, sims2d, _SC_SHARE)
    out = jnp.concatenate([sc_out.reshape(_SC_SHARE, D + 1), tc_out], axis=0)
    return out.reshape(B, i * j, D + 1)


# speedup vs baseline: 6.2301x; 1.0870x over previous
import functools

import jax
import jax.numpy as jnp
from jax import lax
from jax.experimental import pallas as pl
from jax.experimental.pallas import tpu as pltpu
from jax.experimental.pallas import tpu_sc as plsc

_NUM_CORES = 2
_NUM_SUBCORES = 16
_NUM_WORKERS = _NUM_CORES * _NUM_SUBCORES
_LANES = 16
_CHUNK = 16
_SC_SHARE = 7168
_TC_BLK = 512


def _build_sc_kernel(num_loc, K, D, sc_loc):
    per_w = sc_loc // _NUM_WORKERS
    n_chunks = per_w // _CHUNK
    out_row = D + 1

    mesh = plsc.VectorSubcoreMesh(core_axis_name="c", subcore_axis_name="s")

    @functools.partial(
        pl.kernel,
        mesh=mesh,
        compiler_params=pltpu.CompilerParams(needs_layout_passes=False),
        out_type=jax.ShapeDtypeStruct((out_row * sc_loc,), jnp.float32),
        scratch_types=[
            pltpu.VMEM((per_w, K), jnp.float32),
            pltpu.VMEM((per_w,), jnp.int32),
            pltpu.VMEM((_CHUNK, D, K), jnp.float32),
            pltpu.VMEM((_CHUNK, D, K), jnp.float32),
            pltpu.VMEM((out_row * per_w,), jnp.float32),
            pltpu.SemaphoreType.DMA,
            pltpu.SemaphoreType.DMA,
        ],
    )
    def body(dist_hbm, sims_hbm, out_hbm, sims_v, kbuf_v, db0, db1, outbuf_v,
             sem0, sem1):
        wid = lax.axis_index("s") * _NUM_CORES + lax.axis_index("c")
        base_loc = wid * per_w
        iota = lax.iota(jnp.int32, _LANES)
        dbufs = (db0, db1)
        sems = (sem0, sem1)

        def start(c):
            return pltpu.async_copy(
                dist_hbm.at[pl.ds(base_loc + c * _CHUNK, _CHUNK)],
                dbufs[c % 2],
                sems[c % 2],
            )

        pending = start(0)

        pltpu.sync_copy(sims_hbm.at[pl.ds(base_loc, per_w)], sims_v)

        def group_body(g, carry):
            l0 = g * _LANES + iota
            best_val = plsc.load_gather(
                sims_v, [l0, jnp.zeros((_LANES,), jnp.int32)]
            )
            best_k = jnp.zeros((_LANES,), jnp.int32)
            for k in range(1, K):
                v = plsc.load_gather(
                    sims_v, [l0, jnp.full((_LANES,), k, jnp.int32)]
                )
                take = v > best_val
                best_val = jnp.where(take, v, best_val)
                best_k = jnp.where(take, k, best_k)
            plsc.store_scatter(kbuf_v, [l0], best_k)
            outbuf_v[pl.ds(D * per_w + g * _LANES, _LANES)] = (
                best_k.astype(jnp.float32)
            )
            return carry

        lax.fori_loop(0, per_w // _LANES, group_body, 0)

        for c in range(n_chunks):
            nxt = start(c + 1) if c + 1 < n_chunks else None
            pending.wait()
            dbuf = dbufs[c % 2]
            for g in range(_CHUNK // _LANES):
                lb = c * _CHUNK + g * _LANES
                ks = kbuf_v[pl.ds(lb, _LANES)]
                jvec = g * _LANES + iota
                for dd in range(D):
                    val = plsc.load_gather(
                        dbuf, [jvec, jnp.full((_LANES,), dd, jnp.int32), ks]
                    )
                    outbuf_v[pl.ds(dd * per_w + lb, _LANES)] = val
            pending = nxt

        for comp in range(out_row):
            pltpu.sync_copy(
                outbuf_v.at[pl.ds(comp * per_w, per_w)],
                out_hbm.at[pl.ds(comp * sc_loc + base_loc, per_w)],
            )

    return body


def _tc_body(K, D, d_ref, s_ref, o_ref):
    s = s_ref[...]
    ik = lax.broadcasted_iota(jnp.int32, s.shape, 1)
    m = jnp.max(s, axis=-1, keepdims=True)
    am = jnp.min(jnp.where(s == m, ik, K), axis=-1)
    onehot = (ik == am[:, None]).astype(jnp.float32)
    d = d_ref[...]
    resid = jnp.sum(d * onehot[:, None, :], axis=-1)
    o_ref[...] = jnp.concatenate(
        [resid.T, am[None, :].astype(jnp.float32)], axis=0
    )


def _tc_kernel(dist_t, sims2d, start_loc):
    num_loc, D, K = dist_t.shape
    n = num_loc - start_loc
    off = start_loc // _TC_BLK
    return pl.pallas_call(
        functools.partial(_tc_body, K, D),
        grid=(n // _TC_BLK,),
        in_specs=[
            pl.BlockSpec((_TC_BLK, D, K), lambda g: (g + off, 0, 0)),
            pl.BlockSpec((_TC_BLK, K), lambda g: (g + off, 0)),
        ],
        out_specs=pl.BlockSpec((D + 1, _TC_BLK), lambda g: (0, g)),
        out_shape=jax.ShapeDtypeStruct((D + 1, n), jnp.float32),
    )(dist_t, sims2d)


def kernel(distance, similarities):
    B, i, j, K, D = distance.shape
    num_loc = B * i * j
    dist_t = jnp.transpose(distance, (0, 1, 2, 4, 3)).reshape(num_loc, D, K)
    sims2d = similarities.reshape(num_loc, K)
    sc_out = _build_sc_kernel(num_loc, K, D, _SC_SHARE)(dist_t, sims2d)
    tc_out = _tc_kernel(dist_t, sims2d, _SC_SHARE)
    out_t = jnp.concatenate(
        [sc_out.reshape(D + 1, _SC_SHARE), tc_out], axis=1
    )
    return out_t.reshape(D + 1, B, i * j).transpose(1, 2, 0)
